# baseline (speedup 1.0000x reference)
import functools

import jax
import jax.numpy as jnp
from jax import lax
from jax.experimental import pallas as pl
from jax.experimental.pallas import tpu as pltpu
from jax.experimental.pallas import tpu_sc as plsc

B = 16384
D1 = 64
D2 = 128
NC = 2
NS = 16
NW = NC * NS
B_PER_W = B // NW
CHUNK = 128
NCHUNK = B_PER_W // CHUNK

_mesh = plsc.VectorSubcoreMesh(core_axis_name="c", subcore_axis_name="s")


@functools.partial(
    pl.kernel,
    mesh=_mesh,
    compiler_params=pltpu.CompilerParams(use_tc_tiling_on_sc=False),
    out_type=(
        jax.ShapeDtypeStruct((B, D1), jnp.float32),
        jax.ShapeDtypeStruct((B, D2), jnp.float32),
    ),
    scratch_types=[
        pltpu.VMEM((NCHUNK, CHUNK), jnp.int32),
        pltpu.VMEM((B_PER_W, D1), jnp.float32),
        pltpu.VMEM((B_PER_W, D2), jnp.float32),
        pltpu.SemaphoreType.DMA,
    ],
)
def _gather_kernel(ids_hbm, w1_hbm, w2_hbm, out1_hbm, out2_hbm,
                   idx_v, rows1_v, rows2_v, sem):
    wid = lax.axis_index("s") * NC + lax.axis_index("c")
    base = wid * B_PER_W
    pltpu.sync_copy(ids_hbm.at[wid], idx_v)
    copies = []
    for j in range(NCHUNK):
        copies.append(pltpu.async_copy(
            w1_hbm.at[idx_v.at[j]], rows1_v.at[pl.ds(j * CHUNK, CHUNK)], sem))
        copies.append(pltpu.async_copy(
            w2_hbm.at[idx_v.at[j]], rows2_v.at[pl.ds(j * CHUNK, CHUNK)], sem))
    for c in copies:
        c.wait()
    pltpu.sync_copy(rows1_v, out1_hbm.at[pl.ds(base, B_PER_W)])
    pltpu.sync_copy(rows2_v, out2_hbm.at[pl.ds(base, B_PER_W)])


def kernel(instance_ids, W_instance, W_backgrounds):
    ids = jnp.squeeze(instance_ids).astype(jnp.int32)
    ids3 = ids.reshape(NW, NCHUNK, CHUNK)
    return _gather_kernel(ids3, W_instance, W_backgrounds)


# speedup vs baseline: 1.0252x; 1.0252x over previous
import functools

import jax
import jax.numpy as jnp
from jax import lax
from jax.experimental import pallas as pl
from jax.experimental.pallas import tpu as pltpu
from jax.experimental.pallas import tpu_sc as plsc

B = 16384
D1 = 64
D2 = 128
NC = 2
NS = 16
NW = NC * NS
B_PER_W = B // NW
CHUNK = 128
NCHUNK = B_PER_W // CHUNK

_mesh = plsc.VectorSubcoreMesh(core_axis_name="c", subcore_axis_name="s")


def _make_gather(D, tc_tiling):
    @functools.partial(
        pl.kernel,
        mesh=_mesh,
        compiler_params=pltpu.CompilerParams(use_tc_tiling_on_sc=tc_tiling),
        out_type=jax.ShapeDtypeStruct((B, D), jnp.float32),
        scratch_types=[
            pltpu.VMEM((B_PER_W,), jnp.int32),
            pltpu.VMEM((B_PER_W, D), jnp.float32),
            pltpu.SemaphoreType.DMA,
        ],
        name=f"sc_gather_d{D}",
    )
    def _gather(ids_hbm, w_hbm, out_hbm, idx_v, rows_v, sem):
        wid = lax.axis_index("s") * NC + lax.axis_index("c")
        base = wid * B_PER_W
        pltpu.sync_copy(ids_hbm.at[pl.ds(base, B_PER_W)], idx_v)
        copies = []
        for j in range(NCHUNK):
            copies.append(pltpu.async_copy(
                w_hbm.at[idx_v.at[pl.ds(j * CHUNK, CHUNK)]],
                rows_v.at[pl.ds(j * CHUNK, CHUNK)], sem))
        for c in copies:
            c.wait()
        pltpu.sync_copy(rows_v, out_hbm.at[pl.ds(base, B_PER_W)])

    return _gather


_gather_d64 = _make_gather(D1, tc_tiling=False)
_gather_d128 = _make_gather(D2, tc_tiling=True)


def kernel(instance_ids, W_instance, W_backgrounds):
    ids = jnp.squeeze(instance_ids).astype(jnp.int32)
    out1 = _gather_d64(ids, W_instance)
    out2 = _gather_d128(ids, W_backgrounds)
    return (out1, out2)


# speedup vs baseline: 1.4210x; 1.3861x over previous
import functools

import jax
import jax.numpy as jnp
from jax import lax
from jax.experimental import pallas as pl
from jax.experimental.pallas import tpu as pltpu
from jax.experimental.pallas import tpu_sc as plsc

B = 16384
D1 = 64
D2 = 128
NC = 2
NS = 16
NW = NC * NS
B_PER_W = B // NW
CHUNK = 128
NCHUNK = B_PER_W // CHUNK

_mesh = plsc.VectorSubcoreMesh(core_axis_name="c", subcore_axis_name="s")


@functools.partial(
    pl.kernel,
    mesh=_mesh,
    compiler_params=pltpu.CompilerParams(
        use_tc_tiling_on_sc=True, needs_layout_passes=False),
    out_type=(
        jax.ShapeDtypeStruct((B, D1), jnp.float32),
        jax.ShapeDtypeStruct((B, D2), jnp.float32),
    ),
    scratch_types=[
        pltpu.VMEM((B_PER_W,), jnp.int32),
        pltpu.VMEM((B_PER_W, D1), jnp.float32),
        pltpu.VMEM((CHUNK, D2), jnp.float32),
        pltpu.VMEM((CHUNK, D2), jnp.float32),
        pltpu.SemaphoreType.DMA,
        pltpu.SemaphoreType.DMA,
        pltpu.SemaphoreType.DMA,
    ],
    name="sc_dual_gather",
)
def _dual_gather(ids_hbm, w1_hbm, w2_hbm, out1_hbm, out2_hbm,
                 idx_v, rows1_v, rows2a_v, rows2b_v,
                 sem1, sem2a, sem2b):
    wid = lax.axis_index("s") * NC + lax.axis_index("c")
    base = wid * B_PER_W
    pltpu.sync_copy(ids_hbm.at[pl.ds(base, B_PER_W)], idx_v)

    bufs = (rows2a_v, rows2b_v)
    sems = (sem2a, sem2b)

    def _gather2(j):
        return pltpu.async_copy(
            w2_hbm.at[idx_v.at[pl.ds(j * CHUNK, CHUNK)]],
            bufs[j % 2], sems[j % 2])

    inflight = [_gather2(0), _gather2(1)]

    lanes = lax.iota(jnp.int32, NS)

    def _fire(g, _):
        v = idx_v[pl.ds(g * 16, 16)]
        for lane in range(16):
            i = jnp.max(jnp.where(lanes == lane, v, 0))
            pltpu.async_copy(
                w1_hbm.at[pl.ds(i, 1)],
                rows1_v.at[pl.ds(g * 16 + lane, 1)], sem1)
        return _

    lax.fori_loop(0, B_PER_W // 16, _fire, None)

    for j in range(NCHUNK):
        inflight[j % 2].wait()
        pltpu.sync_copy(bufs[j % 2], out2_hbm.at[pl.ds(base + j * CHUNK, CHUNK)])
        if j + 2 < NCHUNK:
            inflight[j % 2] = _gather2(j + 2)

    def _drain(k, _):
        pltpu.make_async_copy(
            w1_hbm.at[pl.ds(0, 1)], rows1_v.at[pl.ds(0, 1)], sem1).wait()
        return _

    lax.fori_loop(0, B_PER_W, _drain, None, unroll=8)
    pltpu.sync_copy(rows1_v, out1_hbm.at[pl.ds(base, B_PER_W)])


def kernel(instance_ids, W_instance, W_backgrounds):
    ids = jnp.squeeze(instance_ids).astype(jnp.int32)
    return _dual_gather(ids, W_instance, W_backgrounds)


# speedup vs baseline: 1.4442x; 1.0163x over previous
import functools

import jax
import jax.numpy as jnp
from jax import lax
from jax.experimental import pallas as pl
from jax.experimental.pallas import tpu as pltpu
from jax.experimental.pallas import tpu_sc as plsc

B = 16384
V = 100000
D1 = 64
D2 = 128
NC = 2
NS = 16
NW = NC * NS
B_PER_W = B // NW
CHUNK2 = 64
NCHUNK2 = B_PER_W // CHUNK2
R_PER_W = D1 // NW
IDX_CHUNK = 2048
NIDX = B // IDX_CHUNK
L = 16

_mesh = plsc.VectorSubcoreMesh(core_axis_name="c", subcore_axis_name="s")


@functools.partial(
    pl.kernel,
    mesh=_mesh,
    compiler_params=pltpu.CompilerParams(
        use_tc_tiling_on_sc=True, needs_layout_passes=False,
        internal_scratch_in_bytes=65536),
    out_type=(
        jax.ShapeDtypeStruct((D1, B), jnp.float32),
        jax.ShapeDtypeStruct((B, D2), jnp.float32),
    ),
    scratch_types=[
        pltpu.VMEM((B_PER_W,), jnp.int32),
        pltpu.VMEM((CHUNK2, D2), jnp.float32),
        pltpu.VMEM((CHUNK2, D2), jnp.float32),
        pltpu.VMEM((1, V), jnp.float32),
        pltpu.VMEM((IDX_CHUNK,), jnp.int32),
        pltpu.VMEM((1, IDX_CHUNK), jnp.float32),
        pltpu.SemaphoreType.DMA,
        pltpu.SemaphoreType.DMA,
        pltpu.SemaphoreType.DMA,
    ],
    name="sc_dual_gather",
)
def _dual_gather(ids_hbm, w1t_hbm, w2_hbm, out1t_hbm, out2_hbm,
                 idx_v, rows2a_v, rows2b_v, wrow_v, idxg_v, og_v,
                 sem1, sem2a, sem2b):
    wid = lax.axis_index("s") * NC + lax.axis_index("c")
    base = wid * B_PER_W
    pltpu.sync_copy(ids_hbm.at[pl.ds(base, B_PER_W)], idx_v)

    bufs = (rows2a_v, rows2b_v)
    sems = (sem2a, sem2b)

    def _gather2(j):
        return pltpu.async_copy(
            w2_hbm.at[idx_v.at[pl.ds(j * CHUNK2, CHUNK2)]],
            bufs[j % 2], sems[j % 2])

    inflight = [_gather2(0), _gather2(1)]

    og_ref = og_v.at[0]

    def _d64_chunk(row_ref, c):
        pltpu.sync_copy(ids_hbm.at[pl.ds(c * IDX_CHUNK, IDX_CHUNK)], idxg_v)

        def _grp(g, _):
            iv = idxg_v[pl.ds(g * L, L)]
            og_ref[pl.ds(g * L, L)] = plsc.load_gather(row_ref, [iv])
            return _

        lax.fori_loop(0, IDX_CHUNK // L, _grp, None, unroll=4)

    for r in range(R_PER_W):
        j = wid * R_PER_W + r
        pltpu.sync_copy(w1t_hbm.at[pl.ds(j, 1)], wrow_v)
        row_ref = wrow_v.at[0]
        for c in range(NIDX):
            _d64_chunk(row_ref, c)
            pltpu.sync_copy(
                og_v, out1t_hbm.at[pl.ds(j, 1), pl.ds(c * IDX_CHUNK, IDX_CHUNK)])
            if r == 0:
                inflight[c % 2].wait()
                pltpu.sync_copy(
                    bufs[c % 2], out2_hbm.at[pl.ds(base + c * CHUNK2, CHUNK2)])
                if c + 2 < NCHUNK2:
                    inflight[c % 2] = _gather2(c + 2)


def kernel(instance_ids, W_instance, W_backgrounds):
    ids = jnp.squeeze(instance_ids).astype(jnp.int32)
    out1t, out2 = _dual_gather(ids, W_instance.T, W_backgrounds)
    return (out1t.T, out2)


# speedup vs baseline: 1.9975x; 1.3831x over previous
import functools

import jax
import jax.numpy as jnp
from jax import lax
from jax.experimental import pallas as pl
from jax.experimental.pallas import tpu as pltpu
from jax.experimental.pallas import tpu_sc as plsc

B = 16384
V = 100000
D1 = 64
D2 = 128
NC = 2
NS = 16
NW = NC * NS
B_PER_W = B // NW
CHUNK2 = 64
NCHUNK2 = B_PER_W // CHUNK2
R_PER_W = D1 // NW
IDX_CHUNK = 2048
NIDX = B // IDX_CHUNK
L = 16

_mesh = plsc.VectorSubcoreMesh(core_axis_name="c", subcore_axis_name="s")


@functools.partial(
    pl.kernel,
    mesh=_mesh,
    compiler_params=pltpu.CompilerParams(
        use_tc_tiling_on_sc=True, needs_layout_passes=False,
        internal_scratch_in_bytes=65536),
    out_type=(
        jax.ShapeDtypeStruct((D1, B), jnp.float32),
        jax.ShapeDtypeStruct((B, D2), jnp.float32),
    ),
    scratch_types=[
        pltpu.VMEM((B_PER_W,), jnp.int32),
        pltpu.VMEM((CHUNK2, D2), jnp.float32),
        pltpu.VMEM((CHUNK2, D2), jnp.float32),
        pltpu.VMEM((1, V), jnp.float32),
        pltpu.VMEM((IDX_CHUNK,), jnp.int32),
        pltpu.VMEM((IDX_CHUNK,), jnp.int32),
        pltpu.VMEM((1, IDX_CHUNK), jnp.float32),
        pltpu.VMEM((1, IDX_CHUNK), jnp.float32),
        pltpu.SemaphoreType.DMA,
        pltpu.SemaphoreType.DMA,
        pltpu.SemaphoreType.DMA,
        pltpu.SemaphoreType.DMA,
        pltpu.SemaphoreType.DMA,
    ],
    name="sc_dual_gather",
)
def _dual_gather(ids_hbm, w1t_hbm, w2_hbm, out1t_hbm, out2_hbm,
                 idx_v, rows2a_v, rows2b_v, wrow_v,
                 idxga_v, idxgb_v, oga_v, ogb_v,
                 sem2a, sem2b, semr, semi, semo):
    wid = lax.axis_index("s") * NC + lax.axis_index("c")
    base = wid * B_PER_W
    j0 = wid * R_PER_W

    hrow = pltpu.async_copy(w1t_hbm.at[pl.ds(j0, 1)], wrow_v, semr)
    pltpu.sync_copy(ids_hbm.at[pl.ds(base, B_PER_W)], idx_v)

    bufs = (rows2a_v, rows2b_v)
    sems = (sem2a, sem2b)

    def _gather2(j):
        return pltpu.async_copy(
            w2_hbm.at[idx_v.at[pl.ds(j * CHUNK2, CHUNK2)]],
            bufs[j % 2], sems[j % 2])

    inflight = [_gather2(0), _gather2(1)]

    idxg = (idxga_v, idxgb_v)
    og = (oga_v, ogb_v)

    def _fetch_idx(c, p):
        return pltpu.async_copy(
            ids_hbm.at[pl.ds(c * IDX_CHUNK, IDX_CHUNK)], idxg[p], semi)

    hidx = [_fetch_idx(0, 0), None]
    row_ref = wrow_v.at[0]
    og_w = [None, None]

    hrow.wait()
    for r in range(R_PER_W):
        j = j0 + r
        if r > 0:
            hrow.wait()
        for c in range(NIDX):
            p = c % 2
            hidx[p].wait()
            if c + 1 < NIDX:
                hidx[(c + 1) % 2] = _fetch_idx(c + 1, (c + 1) % 2)
            elif r + 1 < R_PER_W:
                hidx[0] = _fetch_idx(0, 0)
            if og_w[p] is not None:
                og_w[p].wait()
            ochunk = og[p].at[0]
            ichunk = idxg[p]

            @plsc.parallel_loop(0, IDX_CHUNK // L, unroll=8)
            def _grp(g):
                iv = ichunk[pl.ds(g * L, L)]
                ochunk[pl.ds(g * L, L)] = plsc.load_gather(row_ref, [iv])

            og_w[p] = pltpu.async_copy(
                og[p],
                out1t_hbm.at[pl.ds(j, 1), pl.ds(c * IDX_CHUNK, IDX_CHUNK)],
                semo)
            if r == 0:
                inflight[p].wait()
                pltpu.sync_copy(
                    bufs[p], out2_hbm.at[pl.ds(base + c * CHUNK2, CHUNK2)])
                if c + 2 < NCHUNK2:
                    inflight[p] = _gather2(c + 2)
        if r + 1 < R_PER_W:
            hrow = pltpu.async_copy(
                w1t_hbm.at[pl.ds(j0 + r + 1, 1)], wrow_v, semr)
    og_w[0].wait()
    og_w[1].wait()


def kernel(instance_ids, W_instance, W_backgrounds):
    ids = jnp.squeeze(instance_ids).astype(jnp.int32)
    out1t, out2 = _dual_gather(ids, W_instance.T, W_backgrounds)
    return (out1t.T, out2)
